# baseline (speedup 1.0000x reference)
import functools

import jax
import jax.numpy as jnp
from jax import lax
from jax.experimental import pallas as pl
from jax.experimental.pallas import tpu as pltpu
from jax.experimental.pallas import tpu_sc as plsc

F_IN = 1000000
F_OUT = 16
NF = 26
NB = 16384

NC = 2
NS = 16
NW = NC * NS

BPW = NB // NW
TCW = BPW // 128

_MESH = plsc.VectorSubcoreMesh(core_axis_name="c", subcore_axis_name="s")


@functools.partial(
    pl.kernel,
    out_type=jax.ShapeDtypeStruct((NF * NB,), jnp.int32),
    mesh=_MESH,
    scratch_types=[
        pltpu.VMEM((8, 128), jnp.int32),
    ],
    compiler_params=pltpu.CompilerParams(use_tc_tiling_on_sc=True),
)
def _detile_kernel(xt_hbm, out_hbm, blk):
    wid = lax.axis_index("s") * NC + lax.axis_index("c")
    for bcl in range(TCW):
        bc = wid * TCW + bcl
        for fr in range(4):
            h = 8 if fr < 3 else 2
            pltpu.sync_copy(
                xt_hbm.at[pl.ds(fr * 8, h), pl.ds(bc * 128, 128)],
                blk.at[pl.ds(0, h)])
            for s in range(h):
                f = fr * 8 + s
                pltpu.sync_copy(
                    blk.at[s], out_hbm.at[pl.ds(f * NB + bc * 128, 128)])


def _emb_body(x_hbm, table_hbm, out_hbm, idx_v, buf0, buf1, st0, st1,
              gsem0, gsem1, wsem0, wsem1):
    wid = lax.axis_index("s") * NC + lax.axis_index("c")
    base = wid * BPW
    for f in range(NF):
        pltpu.sync_copy(x_hbm.at[pl.ds(f * NB + base, BPW)], idx_v.at[f])

    bufs = (buf0, buf1)
    stages = (st0, st1)
    gsems = (gsem0, gsem1)
    wsems = (wsem0, wsem1)

    iota = lax.iota(jnp.int32, 16)
    d_idx = [jnp.full((16,), d, jnp.int32) for d in range(F_OUT)]

    def fire_gather(f):
        return pltpu.async_copy(
            table_hbm.at[idx_v.at[f]], bufs[f % 2], gsems[f % 2])

    def transpose_block(p):
        buf, stage = bufs[p], stages[p]

        def body(l16, _):
            b_idx = iota + l16 * 16
            tcl = l16 // 8
            lo = (l16 % 8) * 16
            for d in range(F_OUT):
                v = plsc.load_gather(buf, [b_idx, d_idx[d]])
                stage[d // 8, tcl, d % 8, pl.ds(lo, 16)] = v
            return 0

        lax.fori_loop(0, BPW // 16, body, 0)

    def fire_writeout(f, p):
        return pltpu.async_copy(
            stages[p],
            out_hbm.at[f, :, pl.ds(wid * TCW, TCW)],
            wsems[p])

    gdescs = [None, None]
    wdescs = [None, None]
    gdescs[0] = fire_gather(0)
    gdescs[1] = fire_gather(1)
    for f in range(NF):
        p = f % 2
        gdescs[p].wait()
        if wdescs[p] is not None:
            wdescs[p].wait()
        transpose_block(p)
        if f + 2 < NF:
            gdescs[p] = fire_gather(f + 2)
        wdescs[p] = fire_writeout(f, p)
    wdescs[0].wait()
    wdescs[1].wait()


@functools.partial(
    pl.kernel,
    out_type=jax.ShapeDtypeStruct((NF, 2, 128, 8, 128), jnp.float32),
    mesh=_MESH,
    scratch_types=[
        pltpu.VMEM((NF, BPW), jnp.int32),
        pltpu.VMEM((BPW, F_OUT), jnp.float32),
        pltpu.VMEM((BPW, F_OUT), jnp.float32),
        pltpu.VMEM((2, TCW, 8, 128), jnp.float32),
        pltpu.VMEM((2, TCW, 8, 128), jnp.float32),
        pltpu.SemaphoreType.DMA,
        pltpu.SemaphoreType.DMA,
        pltpu.SemaphoreType.DMA,
        pltpu.SemaphoreType.DMA,
    ],
    compiler_params=pltpu.CompilerParams(
        use_tc_tiling_on_sc=False, needs_layout_passes=False),
)
def _emb_kernel(x_hbm, table_hbm, out_hbm, idx_v, buf0, buf1, st0, st1,
                gsem0, gsem1, wsem0, wsem1):
    _emb_body(x_hbm, table_hbm, out_hbm, idx_v, buf0, buf1, st0, st1,
              gsem0, gsem1, wsem0, wsem1)


def kernel(X, emb_weight):
    xflat = _detile_kernel(X.T)
    out5 = _emb_kernel(xflat, emb_weight)
    return out5.transpose(2, 4, 0, 1, 3).reshape(NB, NF, F_OUT)


# speedup vs baseline: 1.1932x; 1.1932x over previous
import functools

import jax
import jax.numpy as jnp
from jax import lax
from jax.experimental import pallas as pl
from jax.experimental.pallas import tpu as pltpu
from jax.experimental.pallas import tpu_sc as plsc

F_IN = 1000000
F_OUT = 16
NF = 26
NB = 16384

NC = 2
NS = 16
NW = NC * NS

BPW = NB // NW
TCW = BPW // 128

_MESH = plsc.VectorSubcoreMesh(core_axis_name="c", subcore_axis_name="s")


@functools.partial(
    pl.kernel,
    out_type=jax.ShapeDtypeStruct((NF * NB,), jnp.int32),
    mesh=_MESH,
    scratch_types=[
        pltpu.VMEM((8, 128), jnp.int32),
    ],
    compiler_params=pltpu.CompilerParams(
        use_tc_tiling_on_sc=True, needs_layout_passes=False),
)
def _detile_kernel(xt_hbm, out_hbm, blk):
    wid = lax.axis_index("s") * NC + lax.axis_index("c")
    for bcl in range(TCW):
        bc = wid * TCW + bcl
        for fr in range(4):
            h = 8 if fr < 3 else 2
            pltpu.sync_copy(
                xt_hbm.at[pl.ds(fr * 8, h), pl.ds(bc * 128, 128)],
                blk.at[pl.ds(0, h)])
            for s in range(h):
                f = fr * 8 + s
                pltpu.sync_copy(
                    blk.at[s], out_hbm.at[pl.ds(f * NB + bc * 128, 128)])


@functools.partial(
    pl.kernel,
    out_type=jax.ShapeDtypeStruct((F_IN * F_OUT,), jnp.float32),
    mesh=_MESH,
    scratch_types=[
        pltpu.VMEM((F_OUT, 128), jnp.float32),
        pltpu.VMEM((F_OUT, 128), jnp.float32),
        pltpu.VMEM((128 * F_OUT,), jnp.float32),
        pltpu.VMEM((128 * F_OUT,), jnp.float32),
        pltpu.SemaphoreType.DMA,
        pltpu.SemaphoreType.DMA,
        pltpu.SemaphoreType.DMA,
        pltpu.SemaphoreType.DMA,
    ],
    compiler_params=pltpu.CompilerParams(
        use_tc_tiling_on_sc=True, needs_layout_passes=False),
)
def _wtr_kernel(wt_hbm, wtail_hbm, out_hbm, vb0, vb1, ob0, ob1,
                lsem0, lsem1, wsem0, wsem1):
    wid = lax.axis_index("s") * NC + lax.axis_index("c")
    iota16 = lax.iota(jnp.int32, 16) * 16
    NVC = F_IN // 128
    vbufs = (vb0, vb1)
    obufs = (ob0, ob1)
    lsems = (lsem0, lsem1)
    wsems = (wsem0, wsem1)

    def do_tile(vc, h):
        def half(p, vc):
            vb, ob = vbufs[p], obufs[p]
            l0 = pltpu.async_copy(
                wt_hbm.at[pl.ds(0, 8), pl.ds(vc * 128, h)],
                vb.at[pl.ds(0, 8), pl.ds(0, h)], lsems[p])
            l1 = pltpu.async_copy(
                wt_hbm.at[pl.ds(8, 8), pl.ds(vc * 128, h)],
                vb.at[pl.ds(8, 8), pl.ds(0, h)], lsems[p])
            l0.wait()
            l1.wait()
            for l16 in range(h // 16):
                for d in range(F_OUT):
                    val = vb[d, pl.ds(l16 * 16, 16)]
                    plsc.store_scatter(
                        ob, [iota16 + (l16 * 256 + d)], val)
            pltpu.async_copy(
                ob.at[pl.ds(0, h * F_OUT)],
                out_hbm.at[pl.ds(vc * 2048, h * F_OUT)], wsems[p])
        return half

    def body(k2, _):
        for p in range(2):
            vc = wid + (2 * k2 + p) * NW

            @pl.when(vc < NVC)
            def _():
                @pl.when(k2 > 0)
                def _():
                    pltpu.make_async_copy(
                        obufs[p], out_hbm.at[pl.ds(0, 2048)],
                        wsems[p]).wait()
                do_tile(vc, 128)(p, vc)
        return 0

    n_k2 = (NVC + 2 * NW - 1) // (2 * NW)
    lax.fori_loop(0, n_k2, body, 0)
    pltpu.make_async_copy(obufs[0], out_hbm.at[pl.ds(0, 2048)], wsems[0]).wait()
    pltpu.make_async_copy(obufs[1], out_hbm.at[pl.ds(0, 2048)], wsems[1]).wait()

    @pl.when(wid == 0)
    def _():
        pltpu.sync_copy(wtail_hbm, out_hbm.at[pl.ds(NVC * 2048, 64 * F_OUT)])


def _emb_body(x_hbm, table_hbm, out_hbm, idx_v, buf0, buf1, st0, st1,
              gsem0, gsem1, wsem0, wsem1):
    wid = lax.axis_index("s") * NC + lax.axis_index("c")
    base = wid * BPW
    for f in range(NF):
        pltpu.sync_copy(x_hbm.at[pl.ds(f * NB + base, BPW)], idx_v.at[f])

    bufs = (buf0, buf1)
    stages = (st0, st1)
    gsems = (gsem0, gsem1)
    wsems = (wsem0, wsem1)

    iota = lax.iota(jnp.int32, 16)
    doff = ((iota >> 3) << 12) + ((iota & 7) << 7)

    def fire_gather(f):
        return pltpu.async_copy(
            table_hbm.at[idx_v.at[f]], bufs[f % 2], gsems[f % 2])

    def transpose_block(p):
        buf, stage = bufs[p], stages[p]

        def body(l16, _):
            tcl = l16 // 8
            lo = tcl * 1024 + (l16 % 8) * 16
            for j in range(16):
                v = buf[l16 * 16 + j, :]
                plsc.store_scatter(stage, [doff + (lo + j)], v)
            return 0

        lax.fori_loop(0, BPW // 16, body, 0)

    def fire_writeout(f, p):
        d0 = pltpu.async_copy(
            stages[p].at[pl.ds(0, 4096)],
            out_hbm.at[pl.ds(f * 262144 + (wid * TCW) * 1024, 4096)],
            wsems[p])
        d1 = pltpu.async_copy(
            stages[p].at[pl.ds(4096, 4096)],
            out_hbm.at[pl.ds(f * 262144 + (128 + wid * TCW) * 1024, 4096)],
            wsems[p])
        return (d0, d1)

    gdescs = [None, None]
    wdescs = [None, None]
    gdescs[0] = fire_gather(0)
    gdescs[1] = fire_gather(1)
    for f in range(NF):
        p = f % 2
        gdescs[p].wait()
        if wdescs[p] is not None:
            wdescs[p][0].wait()
            wdescs[p][1].wait()
        transpose_block(p)
        if f + 2 < NF:
            gdescs[p] = fire_gather(f + 2)
        wdescs[p] = fire_writeout(f, p)
    for p in range(2):
        wdescs[p][0].wait()
        wdescs[p][1].wait()


@functools.partial(
    pl.kernel,
    out_type=jax.ShapeDtypeStruct((NF * 2 * 128 * 8 * 128,), jnp.float32),
    mesh=_MESH,
    scratch_types=[
        pltpu.VMEM((NF, BPW), jnp.int32),
        pltpu.VMEM((BPW, F_OUT), jnp.float32),
        pltpu.VMEM((BPW, F_OUT), jnp.float32),
        pltpu.VMEM((2 * TCW * 8 * 128,), jnp.float32),
        pltpu.VMEM((2 * TCW * 8 * 128,), jnp.float32),
        pltpu.SemaphoreType.DMA,
        pltpu.SemaphoreType.DMA,
        pltpu.SemaphoreType.DMA,
        pltpu.SemaphoreType.DMA,
    ],
    compiler_params=pltpu.CompilerParams(
        use_tc_tiling_on_sc=False, needs_layout_passes=False),
)
def _emb_kernel(x_hbm, table_hbm, out_hbm, idx_v, buf0, buf1, st0, st1,
                gsem0, gsem1, wsem0, wsem1):
    _emb_body(x_hbm, table_hbm, out_hbm, idx_v, buf0, buf1, st0, st1,
              gsem0, gsem1, wsem0, wsem1)


def kernel(X, emb_weight):
    xflat = _detile_kernel(X.T)
    wtail = emb_weight[(F_IN // 128) * 128:].reshape(-1)
    wflat = _wtr_kernel(emb_weight.T, wtail)
    outflat = _emb_kernel(xflat, wflat.reshape(F_IN, F_OUT))
    out5 = outflat.reshape(NF, 2, 128, 8, 128)
    return out5.transpose(2, 4, 0, 1, 3).reshape(NB, NF, F_OUT)


# speedup vs baseline: 1.9609x; 1.6434x over previous
import functools

import jax
import jax.numpy as jnp
from jax import lax
from jax.experimental import pallas as pl
from jax.experimental.pallas import tpu as pltpu
from jax.experimental.pallas import tpu_sc as plsc

F_IN = 1000000
F_OUT = 16
NF = 26
NB = 16384

NC = 2
NS = 16
NW = NC * NS

BPW = NB // NW
TCW = BPW // 128

_MESH = plsc.VectorSubcoreMesh(core_axis_name="c", subcore_axis_name="s")


@functools.partial(
    pl.kernel,
    out_type=jax.ShapeDtypeStruct((NF * NB,), jnp.int32),
    mesh=_MESH,
    scratch_types=[
        pltpu.VMEM((8, 128), jnp.int32),
    ],
    compiler_params=pltpu.CompilerParams(
        use_tc_tiling_on_sc=True, needs_layout_passes=False),
)
def _detile_kernel(xt_hbm, out_hbm, blk):
    wid = lax.axis_index("s") * NC + lax.axis_index("c")
    for bcl in range(TCW):
        bc = wid * TCW + bcl
        for fr in range(4):
            h = 8 if fr < 3 else 2
            pltpu.sync_copy(
                xt_hbm.at[pl.ds(fr * 8, h), pl.ds(bc * 128, 128)],
                blk.at[pl.ds(0, h)])
            for s in range(h):
                f = fr * 8 + s
                pltpu.sync_copy(
                    blk.at[s], out_hbm.at[pl.ds(f * NB + bc * 128, 128)])


@functools.partial(
    pl.kernel,
    out_type=jax.ShapeDtypeStruct((F_IN * F_OUT,), jnp.float32),
    mesh=_MESH,
    scratch_types=[
        pltpu.VMEM((F_OUT, 128), jnp.float32),
        pltpu.VMEM((F_OUT, 128), jnp.float32),
        pltpu.VMEM((128 * F_OUT,), jnp.float32),
        pltpu.VMEM((128 * F_OUT,), jnp.float32),
        pltpu.SemaphoreType.DMA,
        pltpu.SemaphoreType.DMA,
        pltpu.SemaphoreType.DMA,
        pltpu.SemaphoreType.DMA,
    ],
    compiler_params=pltpu.CompilerParams(
        use_tc_tiling_on_sc=True, needs_layout_passes=False),
)
def _wtr_kernel(wt_hbm, wtail_hbm, out_hbm, vb0, vb1, ob0, ob1,
                lsem0, lsem1, wsem0, wsem1):
    wid = lax.axis_index("s") * NC + lax.axis_index("c")
    iota16 = lax.iota(jnp.int32, 16) * 16
    NVC = F_IN // 128
    vbufs = (vb0, vb1)
    obufs = (ob0, ob1)
    lsems = (lsem0, lsem1)
    wsems = (wsem0, wsem1)

    def fire_loads(p, vc):
        pltpu.async_copy(
            wt_hbm.at[pl.ds(0, 8), pl.ds(vc * 128, 128)],
            vbufs[p].at[pl.ds(0, 8)], lsems[p])
        pltpu.async_copy(
            wt_hbm.at[pl.ds(8, 8), pl.ds(vc * 128, 128)],
            vbufs[p].at[pl.ds(8, 8)], lsems[p])

    def wait_loads(p):
        pltpu.make_async_copy(
            wt_hbm.at[pl.ds(0, 8), pl.ds(0, 128)],
            vbufs[p].at[pl.ds(0, 8)], lsems[p]).wait()
        pltpu.make_async_copy(
            wt_hbm.at[pl.ds(8, 8), pl.ds(0, 128)],
            vbufs[p].at[pl.ds(8, 8)], lsems[p]).wait()

    def transpose_tile(p):
        vb, ob = vbufs[p], obufs[p]
        for l16 in range(8):
            vals = [vb[d, pl.ds(l16 * 16, 16)] for d in range(F_OUT)]
            for d in range(F_OUT):
                plsc.store_scatter(
                    ob, [iota16 + (l16 * 256 + d)], vals[d])

    fire_loads(0, wid)
    fire_loads(1, wid + NW)

    def body(k2, _):
        for p in range(2):
            vc = wid + (2 * k2 + p) * NW

            @pl.when(vc < NVC)
            def _():
                wait_loads(p)
                @pl.when(k2 > 0)
                def _():
                    pltpu.make_async_copy(
                        obufs[p], out_hbm.at[pl.ds(0, 2048)],
                        wsems[p]).wait()
                transpose_tile(p)
                vc_next = vc + 2 * NW

                @pl.when(vc_next < NVC)
                def _():
                    fire_loads(p, vc_next)
                pltpu.async_copy(
                    obufs[p], out_hbm.at[pl.ds(vc * 2048, 2048)], wsems[p])
        return 0

    n_k2 = (NVC + 2 * NW - 1) // (2 * NW)
    lax.fori_loop(0, n_k2, body, 0)
    pltpu.make_async_copy(obufs[0], out_hbm.at[pl.ds(0, 2048)], wsems[0]).wait()
    pltpu.make_async_copy(obufs[1], out_hbm.at[pl.ds(0, 2048)], wsems[1]).wait()

    @pl.when(wid == 0)
    def _():
        pltpu.sync_copy(wtail_hbm, out_hbm.at[pl.ds(NVC * 2048, 64 * F_OUT)])


def _emb_body(x_hbm, table_hbm, out_hbm, idx_v, buf0, buf1, st0, st1,
              gsem0, gsem1, wsem0, wsem1):
    wid = lax.axis_index("s") * NC + lax.axis_index("c")
    base = wid * BPW
    for f in range(NF):
        pltpu.sync_copy(x_hbm.at[pl.ds(f * NB + base, BPW)], idx_v.at[f])

    bufs = (buf0, buf1)
    stages = (st0, st1)
    gsems = (gsem0, gsem1)
    wsems = (wsem0, wsem1)

    iota = lax.iota(jnp.int32, 16)
    doff = ((iota >> 3) << 12) + ((iota & 7) << 7)

    def fire_gather(f):
        return pltpu.async_copy(
            table_hbm.at[idx_v.at[f]], bufs[f % 2], gsems[f % 2])

    def transpose_block(p):
        buf, stage = bufs[p], stages[p]

        def body(l16, _):
            tcl = l16 // 8
            lo = tcl * 1024 + (l16 % 8) * 16
            vals = [buf[l16 * 16 + j, :] for j in range(16)]
            idxs = [doff + (lo + j) for j in range(16)]
            for j in range(16):
                plsc.store_scatter(stage, [idxs[j]], vals[j])
            return 0

        lax.fori_loop(0, BPW // 16, body, 0)

    def fire_writeout(f, p):
        d0 = pltpu.async_copy(
            stages[p].at[pl.ds(0, 4096)],
            out_hbm.at[pl.ds(f * 262144 + (wid * TCW) * 1024, 4096)],
            wsems[p])
        d1 = pltpu.async_copy(
            stages[p].at[pl.ds(4096, 4096)],
            out_hbm.at[pl.ds(f * 262144 + (128 + wid * TCW) * 1024, 4096)],
            wsems[p])
        return (d0, d1)

    gdescs = [None, None]
    wdescs = [None, None]
    gdescs[0] = fire_gather(0)
    gdescs[1] = fire_gather(1)
    for f in range(NF):
        p = f % 2
        gdescs[p].wait()
        if wdescs[p] is not None:
            wdescs[p][0].wait()
            wdescs[p][1].wait()
        transpose_block(p)
        if f + 2 < NF:
            gdescs[p] = fire_gather(f + 2)
        wdescs[p] = fire_writeout(f, p)
    for p in range(2):
        wdescs[p][0].wait()
        wdescs[p][1].wait()


@functools.partial(
    pl.kernel,
    out_type=jax.ShapeDtypeStruct((NF * 2 * 128 * 8 * 128,), jnp.float32),
    mesh=_MESH,
    scratch_types=[
        pltpu.VMEM((NF, BPW), jnp.int32),
        pltpu.VMEM((BPW, F_OUT), jnp.float32),
        pltpu.VMEM((BPW, F_OUT), jnp.float32),
        pltpu.VMEM((2 * TCW * 8 * 128,), jnp.float32),
        pltpu.VMEM((2 * TCW * 8 * 128,), jnp.float32),
        pltpu.SemaphoreType.DMA,
        pltpu.SemaphoreType.DMA,
        pltpu.SemaphoreType.DMA,
        pltpu.SemaphoreType.DMA,
    ],
    compiler_params=pltpu.CompilerParams(
        use_tc_tiling_on_sc=False, needs_layout_passes=False),
)
def _emb_kernel(x_hbm, table_hbm, out_hbm, idx_v, buf0, buf1, st0, st1,
                gsem0, gsem1, wsem0, wsem1):
    _emb_body(x_hbm, table_hbm, out_hbm, idx_v, buf0, buf1, st0, st1,
              gsem0, gsem1, wsem0, wsem1)


def kernel(X, emb_weight):
    xflat = _detile_kernel(X.T)
    wtail = emb_weight[(F_IN // 128) * 128:].reshape(-1)
    wflat = _wtr_kernel(emb_weight.T, wtail)
    outflat = _emb_kernel(xflat, wflat.reshape(F_IN, F_OUT))
    out5 = outflat.reshape(NF, 2, 128, 8, 128)
    return out5.transpose(2, 4, 0, 1, 3).reshape(NB, NF, F_OUT)


# speedup vs baseline: 2.3395x; 1.1931x over previous
import functools

import jax
import jax.numpy as jnp
from jax import lax
from jax.experimental import pallas as pl
from jax.experimental.pallas import tpu as pltpu
from jax.experimental.pallas import tpu_sc as plsc

F_IN = 1000000
F_OUT = 16
NF = 26
NB = 16384

NC = 2
NS = 16
NW = NC * NS

BPW = NB // NW
TCW = BPW // 128

_MESH = plsc.VectorSubcoreMesh(core_axis_name="c", subcore_axis_name="s")


def _pre_body(xt_hbm, wt_hbm, wtail_hbm, xout_hbm, out_hbm,
              blk, vb0, vb1, ob0, ob1, lsem0, lsem1, wsem0, wsem1):
    wid = lax.axis_index("s") * NC + lax.axis_index("c")
    iota16 = lax.iota(jnp.int32, 16) * 16
    NVC2 = F_IN // 256
    vbufs = (vb0, vb1)
    obufs = (ob0, ob1)
    lsems = (lsem0, lsem1)
    wsems = (wsem0, wsem1)

    def fire_loads(p, vc2):
        pltpu.async_copy(
            wt_hbm.at[pl.ds(0, 8), pl.ds(vc2 * 256, 256)],
            vbufs[p].at[pl.ds(0, 8)], lsems[p])
        pltpu.async_copy(
            wt_hbm.at[pl.ds(8, 8), pl.ds(vc2 * 256, 256)],
            vbufs[p].at[pl.ds(8, 8)], lsems[p])

    def wait_loads(p):
        pltpu.make_async_copy(
            wt_hbm.at[pl.ds(0, 8), pl.ds(0, 256)],
            vbufs[p].at[pl.ds(0, 8)], lsems[p]).wait()
        pltpu.make_async_copy(
            wt_hbm.at[pl.ds(8, 8), pl.ds(0, 256)],
            vbufs[p].at[pl.ds(8, 8)], lsems[p]).wait()

    def transpose_tiles(p):
        vb, ob = vbufs[p], obufs[p]
        for l16 in range(16):
            vals = [vb[d, pl.ds(l16 * 16, 16)] for d in range(F_OUT)]
            for d in range(F_OUT):
                plsc.store_scatter(
                    ob, [iota16 + (l16 * 256 + d)], vals[d])

    fire_loads(0, wid)
    fire_loads(1, wid + NW)

    for bcl in range(TCW):
        bc = wid * TCW + bcl
        for fr in range(4):
            h = 8 if fr < 3 else 2
            pltpu.sync_copy(
                xt_hbm.at[pl.ds(fr * 8, h), pl.ds(bc * 128, 128)],
                blk.at[pl.ds(0, h)])
            for s in range(h):
                f = fr * 8 + s
                pltpu.sync_copy(
                    blk.at[s], xout_hbm.at[pl.ds(f * NB + bc * 128, 128)])

    def body(k2, _):
        for p in range(2):
            vc2 = wid + (2 * k2 + p) * NW

            @pl.when(vc2 < NVC2)
            def _():
                wait_loads(p)
                @pl.when(k2 > 0)
                def _():
                    pltpu.make_async_copy(
                        obufs[p], out_hbm.at[pl.ds(0, 4096)],
                        wsems[p]).wait()
                transpose_tiles(p)
                vc2_next = vc2 + 2 * NW

                @pl.when(vc2_next < NVC2)
                def _():
                    fire_loads(p, vc2_next)
                pltpu.async_copy(
                    obufs[p], out_hbm.at[pl.ds(vc2 * 4096, 4096)], wsems[p])
        return 0

    n_k2 = (NVC2 + 2 * NW - 1) // (2 * NW)
    lax.fori_loop(0, n_k2, body, 0)
    pltpu.make_async_copy(obufs[0], out_hbm.at[pl.ds(0, 4096)], wsems[0]).wait()
    pltpu.make_async_copy(obufs[1], out_hbm.at[pl.ds(0, 4096)], wsems[1]).wait()

    @pl.when(wid == 0)
    def _():
        pltpu.sync_copy(
            wtail_hbm, out_hbm.at[pl.ds((F_IN // 128) * 2048, 64 * F_OUT)])


@functools.partial(
    pl.kernel,
    out_type=(
        jax.ShapeDtypeStruct((NF * NB,), jnp.int32),
        jax.ShapeDtypeStruct((F_IN * F_OUT,), jnp.float32),
    ),
    mesh=_MESH,
    scratch_types=[
        pltpu.VMEM((8, 128), jnp.int32),
        pltpu.VMEM((F_OUT, 256), jnp.float32),
        pltpu.VMEM((F_OUT, 256), jnp.float32),
        pltpu.VMEM((256 * F_OUT,), jnp.float32),
        pltpu.VMEM((256 * F_OUT,), jnp.float32),
        pltpu.SemaphoreType.DMA,
        pltpu.SemaphoreType.DMA,
        pltpu.SemaphoreType.DMA,
        pltpu.SemaphoreType.DMA,
    ],
    compiler_params=pltpu.CompilerParams(
        use_tc_tiling_on_sc=True, needs_layout_passes=False),
)
def _pre_kernel(xt_hbm, wt_hbm, wtail_hbm, xout_hbm, out_hbm,
                blk, vb0, vb1, ob0, ob1, lsem0, lsem1, wsem0, wsem1):
    _pre_body(xt_hbm, wt_hbm, wtail_hbm, xout_hbm, out_hbm,
              blk, vb0, vb1, ob0, ob1, lsem0, lsem1, wsem0, wsem1)


def _emb_body(x_hbm, table_hbm, out_hbm, idx_v, buf0, buf1, st0, st1,
              gsem0, gsem1, wsem0, wsem1):
    wid = lax.axis_index("s") * NC + lax.axis_index("c")
    base = wid * BPW
    for f in range(NF):
        pltpu.sync_copy(x_hbm.at[pl.ds(f * NB + base, BPW)], idx_v.at[f])

    bufs = (buf0, buf1)
    stages = (st0, st1)
    gsems = (gsem0, gsem1)
    wsems = (wsem0, wsem1)

    iota = lax.iota(jnp.int32, 16)
    doff = ((iota >> 3) << 12) + ((iota & 7) << 7)

    def fire_gather(f):
        return pltpu.async_copy(
            table_hbm.at[idx_v.at[f]], bufs[f % 2], gsems[f % 2])

    def transpose_block(p):
        buf, stage = bufs[p], stages[p]

        def body(l16, _):
            tcl = l16 // 8
            lo = tcl * 1024 + (l16 % 8) * 16
            vals = [buf[l16 * 16 + j, :] for j in range(16)]
            idxs = [doff + (lo + j) for j in range(16)]
            for j in range(16):
                plsc.store_scatter(stage, [idxs[j]], vals[j])
            return 0

        lax.fori_loop(0, BPW // 16, body, 0)

    def fire_writeout(f, p):
        d0 = pltpu.async_copy(
            stages[p].at[pl.ds(0, 4096)],
            out_hbm.at[pl.ds(f * 262144 + (wid * TCW) * 1024, 4096)],
            wsems[p])
        d1 = pltpu.async_copy(
            stages[p].at[pl.ds(4096, 4096)],
            out_hbm.at[pl.ds(f * 262144 + (128 + wid * TCW) * 1024, 4096)],
            wsems[p])
        return (d0, d1)

    gdescs = [None, None]
    wdescs = [None, None]
    gdescs[0] = fire_gather(0)
    gdescs[1] = fire_gather(1)
    for f in range(NF):
        p = f % 2
        gdescs[p].wait()
        if wdescs[p] is not None:
            wdescs[p][0].wait()
            wdescs[p][1].wait()
        transpose_block(p)
        if f + 2 < NF:
            gdescs[p] = fire_gather(f + 2)
        wdescs[p] = fire_writeout(f, p)
    for p in range(2):
        wdescs[p][0].wait()
        wdescs[p][1].wait()


@functools.partial(
    pl.kernel,
    out_type=jax.ShapeDtypeStruct((NF * 2 * 128 * 8 * 128,), jnp.float32),
    mesh=_MESH,
    scratch_types=[
        pltpu.VMEM((NF, BPW), jnp.int32),
        pltpu.VMEM((BPW, F_OUT), jnp.float32),
        pltpu.VMEM((BPW, F_OUT), jnp.float32),
        pltpu.VMEM((2 * TCW * 8 * 128,), jnp.float32),
        pltpu.VMEM((2 * TCW * 8 * 128,), jnp.float32),
        pltpu.SemaphoreType.DMA,
        pltpu.SemaphoreType.DMA,
        pltpu.SemaphoreType.DMA,
        pltpu.SemaphoreType.DMA,
    ],
    compiler_params=pltpu.CompilerParams(
        use_tc_tiling_on_sc=False, needs_layout_passes=False),
)
def _emb_kernel(x_hbm, table_hbm, out_hbm, idx_v, buf0, buf1, st0, st1,
                gsem0, gsem1, wsem0, wsem1):
    _emb_body(x_hbm, table_hbm, out_hbm, idx_v, buf0, buf1, st0, st1,
              gsem0, gsem1, wsem0, wsem1)


def kernel(X, emb_weight):
    wtail = emb_weight[(F_IN // 128) * 128:].reshape(-1)
    xflat, wflat = _pre_kernel(X.T, emb_weight.T, wtail)
    outflat = _emb_kernel(xflat, wflat.reshape(F_IN, F_OUT))
    out5 = outflat.reshape(NF, 2, 128, 8, 128)
    return out5.transpose(2, 4, 0, 1, 3).reshape(NB, NF, F_OUT)
